# baseline (speedup 1.0000x reference)
import functools

import jax
import jax.numpy as jnp
from jax import lax
from jax.experimental import pallas as pl
from jax.experimental.pallas import tpu as pltpu
from jax.experimental.pallas import tpu_sc as plsc

BATCH = 16384
EMBED_DIM = 32
NEG_NUM = 4
NUM_WORKERS = 32
B_PER_W = BATCH // NUM_WORKERS
CHUNK = 128
UP_CHUNKS = B_PER_W // CHUNK
NEG_CHUNKS = B_PER_W * NEG_NUM // CHUNK
GROUPS = B_PER_W // 16
OUT_COLS = 2 * NEG_NUM


def _bpr_body(user_idx_hbm, pos_idx_hbm, neg_idx_hbm, utab_hbm, itab_hbm,
              out_hbm, idx_u, idx_p, idx_n, urows, prows, nrows, outv, sem):
    c = lax.axis_index("c")
    s = lax.axis_index("s")
    wid = s * 2 + c
    base = wid * B_PER_W

    pltpu.sync_copy(user_idx_hbm.at[wid], idx_u)
    pltpu.sync_copy(pos_idx_hbm.at[wid], idx_p)
    pltpu.sync_copy(neg_idx_hbm.at[wid], idx_n)

    copies = []
    for j in range(UP_CHUNKS):
        copies.append(pltpu.async_copy(
            utab_hbm.at[idx_u.at[j]], urows.at[pl.ds(j * CHUNK, CHUNK)], sem))
        copies.append(pltpu.async_copy(
            itab_hbm.at[idx_p.at[j]], prows.at[pl.ds(j * CHUNK, CHUNK)], sem))
    for j in range(NEG_CHUNKS):
        copies.append(pltpu.async_copy(
            itab_hbm.at[idx_n.at[j]], nrows.at[pl.ds(j * CHUNK, CHUNK)], sem))
    for cp in copies:
        cp.wait()

    lane = lax.iota(jnp.int32, 16)

    def group(g, carry):
        row = g * 16 + lane
        rown = [row * NEG_NUM + j for j in range(NEG_NUM)]
        accp = jnp.zeros((16,), jnp.float32)
        accn = [jnp.zeros((16,), jnp.float32) for _ in range(NEG_NUM)]
        for d in range(EMBED_DIM):
            col = jnp.full((16,), d, jnp.int32)
            u = plsc.load_gather(urows, [row, col])
            p = plsc.load_gather(prows, [row, col])
            accp = accp + u * p
            for j in range(NEG_NUM):
                n = plsc.load_gather(nrows, [rown[j], col])
                accn[j] = accn[j] + u * n
        for cc in range(NEG_NUM):
            plsc.store_scatter(outv, [row, jnp.full((16,), cc, jnp.int32)], accp)
        for j in range(NEG_NUM):
            plsc.store_scatter(
                outv, [row, jnp.full((16,), NEG_NUM + j, jnp.int32)], accn[j])
        return carry

    lax.fori_loop(0, GROUPS, group, 0)

    pltpu.sync_copy(outv, out_hbm.at[pl.ds(base, B_PER_W)])


@jax.jit
def _bpr(user_c, pos_c, neg_c, user_table, item_table):
    mesh = plsc.VectorSubcoreMesh(core_axis_name="c", subcore_axis_name="s")
    f = pl.kernel(
        _bpr_body,
        out_type=jax.ShapeDtypeStruct((BATCH, OUT_COLS), jnp.float32),
        mesh=mesh,
        scratch_types=[
            pltpu.VMEM((UP_CHUNKS, CHUNK), jnp.int32),
            pltpu.VMEM((UP_CHUNKS, CHUNK), jnp.int32),
            pltpu.VMEM((NEG_CHUNKS, CHUNK), jnp.int32),
            pltpu.VMEM((B_PER_W, EMBED_DIM), jnp.float32),
            pltpu.VMEM((B_PER_W, EMBED_DIM), jnp.float32),
            pltpu.VMEM((B_PER_W * NEG_NUM, EMBED_DIM), jnp.float32),
            pltpu.VMEM((B_PER_W, OUT_COLS), jnp.float32),
            pltpu.SemaphoreType.DMA,
        ],
        compiler_params=pltpu.CompilerParams(
            needs_layout_passes=False, use_tc_tiling_on_sc=False),
    )
    return f(user_c, pos_c, neg_c, user_table, item_table)


def kernel(user, pos_item, neg_item, user_table, item_table):
    user_c = user.reshape(NUM_WORKERS, UP_CHUNKS, CHUNK).astype(jnp.int32)
    pos_c = pos_item.reshape(NUM_WORKERS, UP_CHUNKS, CHUNK).astype(jnp.int32)
    neg_c = neg_item.reshape(NUM_WORKERS, NEG_CHUNKS, CHUNK).astype(jnp.int32)
    return _bpr(user_c, pos_c, neg_c, user_table, item_table)
